# Initial kernel scaffold; baseline (speedup 1.0000x reference)
import jax
import jax.numpy as jnp
from jax.experimental import pallas as pl


def kernel(idx, table):
    raise NotImplementedError("write your pallas kernel here")


# speedup vs baseline: 1.5450x; 1.5450x over previous
import functools

import jax
import jax.numpy as jnp
from jax import lax
from jax.experimental import pallas as pl
from jax.experimental.pallas import tpu as pltpu
from jax.experimental.pallas import tpu_sc as plsc

NC = 2
NS = 16
NW = NC * NS

EMBED_DIM = 32
CHUNK = 1024
SUB = 128


def _gather_body(b_per_w, n_chunks, idx_hbm, table_hbm, out_hbm,
                 idx_v, rows_v, sem):
    wid = lax.axis_index("s") * NC + lax.axis_index("c")
    base = wid * b_per_w
    for i in range(n_chunks):
        off = base + i * CHUNK
        pltpu.sync_copy(idx_hbm.at[pl.ds(off, CHUNK)], idx_v)
        for j in range(CHUNK // SUB):
            pltpu.async_copy(
                table_hbm.at[idx_v.at[pl.ds(j * SUB, SUB)]],
                rows_v.at[pl.ds(j * SUB, SUB)],
                sem,
            )
        pltpu.make_async_copy(
            table_hbm.at[idx_v], rows_v, sem).wait()
        pltpu.sync_copy(rows_v, out_hbm.at[pl.ds(off, CHUNK)])


@functools.partial(jax.jit, static_argnames=())
def _sc_gather(idx_flat, table):
    b = idx_flat.shape[0]
    assert b % (NW * CHUNK) == 0
    b_per_w = b // NW
    n_chunks = b_per_w // CHUNK
    mesh = plsc.VectorSubcoreMesh(
        core_axis_name="c", subcore_axis_name="s",
        num_cores=NC, num_subcores=NS)
    body = functools.partial(_gather_body, b_per_w, n_chunks)
    return pl.kernel(
        body,
        out_type=jax.ShapeDtypeStruct((b, EMBED_DIM), jnp.float32),
        mesh=mesh,
        scratch_types=[
            pltpu.VMEM((CHUNK,), jnp.int32),
            pltpu.VMEM((CHUNK, EMBED_DIM), jnp.float32),
            pltpu.SemaphoreType.DMA,
        ],
        compiler_params=pltpu.CompilerParams(use_tc_tiling_on_sc=False),
    )(idx_flat, table)


def kernel(idx, table):
    bsz, fields = idx.shape
    flat = idx.reshape(bsz * fields).astype(jnp.int32)
    out = _sc_gather(flat, table)
    return out.reshape(bsz, fields, EMBED_DIM)


# speedup vs baseline: 1.5756x; 1.0198x over previous
import functools

import jax
import jax.numpy as jnp
from jax import lax
from jax.experimental import pallas as pl
from jax.experimental.pallas import tpu as pltpu
from jax.experimental.pallas import tpu_sc as plsc

NC = 2
NS = 16
NW = NC * NS

EMBED_DIM = 32
CHUNK = 1664
SUB = 128
NSUB = CHUNK // SUB


def _gather_body(b_per_w, n_chunks, idx_hbm, table_hbm, out_hbm,
                 idx_v, rows0, rows1, gsem0, gsem1, osem0, osem1):
    wid = lax.axis_index("s") * NC + lax.axis_index("c")
    base = wid * b_per_w
    rows = (rows0, rows1)
    gsem = (gsem0, gsem1)
    osem = (osem0, osem1)

    pltpu.sync_copy(idx_hbm.at[pl.ds(base, b_per_w)], idx_v)

    def fire_gather(i, b):
        for j in range(NSUB):
            pltpu.async_copy(
                table_hbm.at[idx_v.at[pl.ds(i * CHUNK + j * SUB, SUB)]],
                rows[b].at[pl.ds(j * SUB, SUB)],
                gsem[b])

    def drain_gather(b):
        pltpu.make_async_copy(
            table_hbm.at[idx_v.at[pl.ds(0, CHUNK)]], rows[b], gsem[b]).wait()

    def fire_out(i, b):
        pltpu.async_copy(
            rows[b], out_hbm.at[pl.ds(base + i * CHUNK, CHUNK)], osem[b])

    def drain_out(i, b):
        pltpu.make_async_copy(
            rows[b], out_hbm.at[pl.ds(base + i * CHUNK, CHUNK)],
            osem[b]).wait()

    fire_gather(0, 0)
    for i in range(1, n_chunks):
        b_prev, b_cur = (i - 1) % 2, i % 2
        if i >= 2:
            drain_out(i - 2, b_cur)
        fire_gather(i, b_cur)
        drain_gather(b_prev)
        fire_out(i - 1, b_prev)
    last = n_chunks - 1
    drain_gather(last % 2)
    fire_out(last, last % 2)
    if n_chunks >= 2:
        drain_out(n_chunks - 2, (n_chunks - 2) % 2)
    drain_out(last, last % 2)


@jax.jit
def _sc_gather(idx_flat, table):
    b = idx_flat.shape[0]
    assert b % (NW * CHUNK) == 0
    b_per_w = b // NW
    n_chunks = b_per_w // CHUNK
    mesh = plsc.VectorSubcoreMesh(
        core_axis_name="c", subcore_axis_name="s",
        num_cores=NC, num_subcores=NS)
    body = functools.partial(_gather_body, b_per_w, n_chunks)
    return pl.kernel(
        body,
        out_type=jax.ShapeDtypeStruct((b, EMBED_DIM), jnp.float32),
        mesh=mesh,
        scratch_types=[
            pltpu.VMEM((b_per_w,), jnp.int32),
            pltpu.VMEM((CHUNK, EMBED_DIM), jnp.float32),
            pltpu.VMEM((CHUNK, EMBED_DIM), jnp.float32),
            pltpu.SemaphoreType.DMA,
            pltpu.SemaphoreType.DMA,
            pltpu.SemaphoreType.DMA,
            pltpu.SemaphoreType.DMA,
        ],
        compiler_params=pltpu.CompilerParams(use_tc_tiling_on_sc=False),
    )(idx_flat, table)


def kernel(idx, table):
    bsz, fields = idx.shape
    flat = idx.reshape(bsz * fields).astype(jnp.int32)
    out = _sc_gather(flat, table)
    return out.reshape(bsz, fields, EMBED_DIM)
